# Initial kernel scaffold; baseline (speedup 1.0000x reference)
import jax
import jax.numpy as jnp
from jax.experimental import pallas as pl


def kernel(x, codebook_s0, codebook_s1, codebook_m0, codebook_m1, codebook_c0, codebook_c1):
    raise NotImplementedError("write your pallas kernel here")


# speedup vs baseline: 1.5165x; 1.5165x over previous
import functools

import jax
import jax.numpy as jnp
from jax.experimental import pallas as pl
from jax.experimental.pallas import tpu as pltpu

BETA = 0.25
K = 1024


def _vq_stage(r, cb):
    tb = r.shape[0]
    cn = jnp.sum(cb * cb, axis=1)[None, :]
    rn = jnp.sum(r * r, axis=1, keepdims=True)
    g = jax.lax.dot_general(r, cb, (((1,), (1,)), ((), ())),
                            preferred_element_type=jnp.float32)
    d = rn + cn - 2.0 * g
    m = jnp.min(d, axis=1, keepdims=True)
    iota_k = jax.lax.broadcasted_iota(jnp.int32, (tb, K), 1)
    idx = jnp.min(jnp.where(d == m, iota_k, K), axis=1, keepdims=True)
    onehot = (iota_k == idx).astype(jnp.float32)
    xq = jax.lax.dot_general(onehot, cb, (((1,), (0,)), ((), ())),
                             preferred_element_type=jnp.float32)
    xq_st = r + (xq - r)
    r_new = r - xq_st
    lsum = jnp.sum((xq - r) ** 2)
    return r_new, xq_st, idx, lsum


def _rvq_body(x_ref, s0_ref, s1_ref, m0_ref, m1_ref, c0_ref, c1_ref,
              sem_ref, col_ref, loss_ref,
              is0_ref, is1_ref, im0_ref, im1_ref, ic0_ref, ic1_ref,
              *, half):
    i = pl.program_id(0)

    @pl.when(i == 0)
    def _init():
        loss_ref[...] = jnp.zeros_like(loss_ref)

    x = x_ref[...]

    r, xq0, i0, l0 = _vq_stage(x, s0_ref[...])
    r, xq1, i1, l1 = _vq_stage(r, s1_ref[...])
    xq_sh = xq0 + xq1

    rs = r[:, :half]
    rs, xm0, i2, l2 = _vq_stage(rs, m0_ref[...])
    rs, xm1, i3, l3 = _vq_stage(rs, m1_ref[...])
    sem_ref[...] = (xq_sh[:, :half] + xm0) + xm1

    rc = r[:, half:]
    rc, xc0, i4, l4 = _vq_stage(rc, c0_ref[...])
    rc, xc1, i5, l5 = _vq_stage(rc, c1_ref[...])
    col_ref[...] = (xq_sh[:, half:] + xc0) + xc1

    is0_ref[...] = i0
    is1_ref[...] = i1
    im0_ref[...] = i2
    im1_ref[...] = i3
    ic0_ref[...] = i4
    ic1_ref[...] = i5

    row = jax.lax.broadcasted_iota(jnp.int32, loss_ref.shape, 0)
    acc = jnp.zeros(loss_ref.shape, jnp.float32)
    for k, s in enumerate((l0, l1, l2, l3, l4, l5)):
        acc = acc + jnp.where(row == k, s, 0.0)
    loss_ref[...] += acc


@functools.partial(jax.jit, static_argnames=())
def kernel(x, codebook_s0, codebook_s1, codebook_m0, codebook_m1,
           codebook_c0, codebook_c1):
    b, d = x.shape
    half = d // 2
    tb = 256
    grid = b // tb

    cb_spec_full = pl.BlockSpec((K, d), lambda i: (0, 0))
    cb_spec_half = pl.BlockSpec((K, half), lambda i: (0, 0))

    out_shapes = (
        jax.ShapeDtypeStruct((b, half), jnp.float32),
        jax.ShapeDtypeStruct((b, half), jnp.float32),
        jax.ShapeDtypeStruct((8, 128), jnp.float32),
        jax.ShapeDtypeStruct((b, 1), jnp.int32),
        jax.ShapeDtypeStruct((b, 1), jnp.int32),
        jax.ShapeDtypeStruct((b, 1), jnp.int32),
        jax.ShapeDtypeStruct((b, 1), jnp.int32),
        jax.ShapeDtypeStruct((b, 1), jnp.int32),
        jax.ShapeDtypeStruct((b, 1), jnp.int32),
    )
    half_spec = pl.BlockSpec((tb, half), lambda i: (i, 0))
    idx_spec = pl.BlockSpec((tb, 1), lambda i: (i, 0))
    out_specs = (
        half_spec, half_spec,
        pl.BlockSpec((8, 128), lambda i: (0, 0)),
        idx_spec, idx_spec, idx_spec, idx_spec, idx_spec, idx_spec,
    )

    outs = pl.pallas_call(
        functools.partial(_rvq_body, half=half),
        grid=(grid,),
        in_specs=[
            pl.BlockSpec((tb, d), lambda i: (i, 0)),
            cb_spec_full, cb_spec_full,
            cb_spec_half, cb_spec_half, cb_spec_half, cb_spec_half,
        ],
        out_specs=out_specs,
        out_shape=out_shapes,
        compiler_params=pltpu.CompilerParams(
            dimension_semantics=("arbitrary",),
        ),
    )(x, codebook_s0, codebook_s1, codebook_m0, codebook_m1,
      codebook_c0, codebook_c1)

    sem_xq, col_xq, loss_sums, i0, i1, i2, i3, i4, i5 = outs

    sums = loss_sums[:6, 0]
    denoms = jnp.array([b * d, b * d, b * half, b * half, b * half, b * half],
                       jnp.float32)
    means = sums / denoms
    losses = BETA * means + means
    mean_losses = jnp.mean(losses)

    semantic_indices = jnp.concatenate([i0, i1, i2, i3], axis=1)
    collaborate_indices = jnp.concatenate([i0, i1, i4, i5], axis=1)
    return (sem_xq, col_xq, mean_losses, semantic_indices, collaborate_indices)


# speedup vs baseline: 1.5516x; 1.0231x over previous
import functools

import jax
import jax.numpy as jnp
from jax.experimental import pallas as pl
from jax.experimental.pallas import tpu as pltpu

BETA = 0.25
K = 1024


def _vq_stage(r, cb, cn):
    tb = r.shape[0]
    rn = jnp.sum(r * r, axis=1, keepdims=True)
    g = jax.lax.dot_general(r, cb, (((1,), (1,)), ((), ())),
                            preferred_element_type=jnp.float32)
    d = rn + cn - 2.0 * g
    m = jnp.min(d, axis=1, keepdims=True)
    iota_k = jax.lax.broadcasted_iota(jnp.int32, (tb, K), 1)
    idx = jnp.min(jnp.where(d == m, iota_k, K), axis=1, keepdims=True)
    onehot = (iota_k == idx).astype(jnp.float32)
    xq = jax.lax.dot_general(onehot, cb, (((1,), (0,)), ((), ())),
                             preferred_element_type=jnp.float32)
    xq_st = r + (xq - r)
    r_new = r - xq_st
    lsum = jnp.sum((xq - r) ** 2)
    return r_new, xq_st, idx, lsum


def _rvq_body(x_ref, s0_ref, s1_ref, m0_ref, m1_ref, c0_ref, c1_ref,
              sem_ref, col_ref, loss_ref,
              is0_ref, is1_ref, im0_ref, im1_ref, ic0_ref, ic1_ref,
              cn_ref, *, half):
    i = pl.program_id(0)

    @pl.when(i == 0)
    def _init():
        loss_ref[...] = jnp.zeros_like(loss_ref)
        cn_ref[0:1, :] = jnp.sum(s0_ref[...] * s0_ref[...], axis=1)[None, :]
        cn_ref[1:2, :] = jnp.sum(s1_ref[...] * s1_ref[...], axis=1)[None, :]
        cn_ref[2:3, :] = jnp.sum(m0_ref[...] * m0_ref[...], axis=1)[None, :]
        cn_ref[3:4, :] = jnp.sum(m1_ref[...] * m1_ref[...], axis=1)[None, :]
        cn_ref[4:5, :] = jnp.sum(c0_ref[...] * c0_ref[...], axis=1)[None, :]
        cn_ref[5:6, :] = jnp.sum(c1_ref[...] * c1_ref[...], axis=1)[None, :]

    x = x_ref[...]

    r, xq0, i0, l0 = _vq_stage(x, s0_ref[...], cn_ref[0:1, :])
    r, xq1, i1, l1 = _vq_stage(r, s1_ref[...], cn_ref[1:2, :])
    xq_sh = xq0 + xq1

    rs = r[:, :half]
    rs, xm0, i2, l2 = _vq_stage(rs, m0_ref[...], cn_ref[2:3, :])
    rs, xm1, i3, l3 = _vq_stage(rs, m1_ref[...], cn_ref[3:4, :])
    sem_ref[...] = (xq_sh[:, :half] + xm0) + xm1

    rc = r[:, half:]
    rc, xc0, i4, l4 = _vq_stage(rc, c0_ref[...], cn_ref[4:5, :])
    rc, xc1, i5, l5 = _vq_stage(rc, c1_ref[...], cn_ref[5:6, :])
    col_ref[...] = (xq_sh[:, half:] + xc0) + xc1

    is0_ref[...] = i0
    is1_ref[...] = i1
    im0_ref[...] = i2
    im1_ref[...] = i3
    ic0_ref[...] = i4
    ic1_ref[...] = i5

    row = jax.lax.broadcasted_iota(jnp.int32, loss_ref.shape, 0)
    acc = jnp.zeros(loss_ref.shape, jnp.float32)
    for k, s in enumerate((l0, l1, l2, l3, l4, l5)):
        acc = acc + jnp.where(row == k, s, 0.0)
    loss_ref[...] += acc


@functools.partial(jax.jit, static_argnames=())
def kernel(x, codebook_s0, codebook_s1, codebook_m0, codebook_m1,
           codebook_c0, codebook_c1):
    b, d = x.shape
    half = d // 2
    tb = 256
    grid = b // tb

    cb_spec_full = pl.BlockSpec((K, d), lambda i: (0, 0))
    cb_spec_half = pl.BlockSpec((K, half), lambda i: (0, 0))

    out_shapes = (
        jax.ShapeDtypeStruct((b, half), jnp.float32),
        jax.ShapeDtypeStruct((b, half), jnp.float32),
        jax.ShapeDtypeStruct((8, 128), jnp.float32),
        jax.ShapeDtypeStruct((b, 1), jnp.int32),
        jax.ShapeDtypeStruct((b, 1), jnp.int32),
        jax.ShapeDtypeStruct((b, 1), jnp.int32),
        jax.ShapeDtypeStruct((b, 1), jnp.int32),
        jax.ShapeDtypeStruct((b, 1), jnp.int32),
        jax.ShapeDtypeStruct((b, 1), jnp.int32),
    )
    half_spec = pl.BlockSpec((tb, half), lambda i: (i, 0))
    idx_spec = pl.BlockSpec((tb, 1), lambda i: (i, 0))
    out_specs = (
        half_spec, half_spec,
        pl.BlockSpec((8, 128), lambda i: (0, 0)),
        idx_spec, idx_spec, idx_spec, idx_spec, idx_spec, idx_spec,
    )

    outs = pl.pallas_call(
        functools.partial(_rvq_body, half=half),
        grid=(grid,),
        in_specs=[
            pl.BlockSpec((tb, d), lambda i: (i, 0)),
            cb_spec_full, cb_spec_full,
            cb_spec_half, cb_spec_half, cb_spec_half, cb_spec_half,
        ],
        out_specs=out_specs,
        out_shape=out_shapes,
        scratch_shapes=[pltpu.VMEM((8, K), jnp.float32)],
        compiler_params=pltpu.CompilerParams(
            dimension_semantics=("arbitrary",),
        ),
    )(x, codebook_s0, codebook_s1, codebook_m0, codebook_m1,
      codebook_c0, codebook_c1)

    sem_xq, col_xq, loss_sums, i0, i1, i2, i3, i4, i5 = outs

    sums = loss_sums[:6, 0]
    denoms = jnp.array([b * d, b * d, b * half, b * half, b * half, b * half],
                       jnp.float32)
    means = sums / denoms
    losses = BETA * means + means
    mean_losses = jnp.mean(losses)

    semantic_indices = jnp.concatenate([i0, i1, i2, i3], axis=1)
    collaborate_indices = jnp.concatenate([i0, i1, i4, i5], axis=1)
    return (sem_xq, col_xq, mean_losses, semantic_indices, collaborate_indices)
